# baseline (speedup 1.0000x reference)
import functools

import jax
import jax.numpy as jnp
from jax import lax
from jax.experimental import pallas as pl
from jax.experimental.pallas import tpu as pltpu
from jax.experimental.pallas import tpu_sc as plsc

VOCAB = 1000000
D = 32
SCALE = float(D) ** 0.5

NC = 2
NS = 16
NW = NC * NS

GROUP = 128
NG = 50
G_PER_CHUNK = 5
NCHUNK = NG // G_PER_CHUNK


def _body(idx_hbm, table_hbm, out_hbm, idx_v, buf0, buf1, gs0, gs1, ss0, ss1):
    wid = lax.axis_index("s") * NC + lax.axis_index("c")
    gbase = wid * NG

    pltpu.sync_copy(idx_hbm.at[wid], idx_v)

    bufs = (buf0, buf1)
    gsems = (gs0, gs1)
    ssems = (ss0, ss1)
    gather_handles = [None, None]
    scatter_handles = [None, None]

    def fire_gather(c):
        b = c % 2
        hs = []
        for j in range(G_PER_CHUNK):
            g = c * G_PER_CHUNK + j
            hs.append(
                pltpu.async_copy(
                    table_hbm.at[idx_v.at[g]], bufs[b].at[j], gsems[b]
                )
            )
        gather_handles[b] = hs

    def compute_and_scatter(c):
        b = c % 2
        buf = bufs[b]
        for h in gather_handles[b]:
            h.wait()
        for j in range(G_PER_CHUNK):
            g = c * G_PER_CHUNK + j

            def rowblk(t, _, j=j, g=g, buf=buf):
                iv = idx_v[g, pl.ds(t * 16, 16)]
                mvec = jnp.where(iv == 0, 0.0, SCALE)
                for r8 in range(16):
                    r = t * 16 + r8
                    m = mvec[r8]
                    buf[j, r, pl.ds(0, 16)] = buf[j, r, pl.ds(0, 16)] * m
                    buf[j, r, pl.ds(16, 16)] = buf[j, r, pl.ds(16, 16)] * m
                return 0

            lax.fori_loop(0, GROUP // 16, rowblk, 0)
        scatter_handles[b] = pltpu.async_copy(
            buf, out_hbm.at[pl.ds(gbase + c * G_PER_CHUNK, G_PER_CHUNK)], ssems[b]
        )

    fire_gather(0)
    for c in range(NCHUNK):
        if c + 1 < NCHUNK:
            b = (c + 1) % 2
            if scatter_handles[b] is not None:
                scatter_handles[b].wait()
            fire_gather(c + 1)
        compute_and_scatter(c)
    for b in range(2):
        if scatter_handles[b] is not None:
            scatter_handles[b].wait()


@jax.jit
def kernel(inputs, shared_weights):
    S, T = inputs.shape
    nrows = S * T // GROUP
    idx = inputs.reshape(NW, NG, GROUP).astype(jnp.int32)

    mesh = plsc.VectorSubcoreMesh(core_axis_name="c", subcore_axis_name="s")
    run = functools.partial(
        pl.kernel,
        mesh=mesh,
        out_type=jax.ShapeDtypeStruct((nrows, GROUP, D), jnp.float32),
        scratch_types=[
            pltpu.VMEM((NG, GROUP), jnp.int32),
            pltpu.VMEM((G_PER_CHUNK, GROUP, D), jnp.float32),
            pltpu.VMEM((G_PER_CHUNK, GROUP, D), jnp.float32),
            pltpu.SemaphoreType.DMA,
            pltpu.SemaphoreType.DMA,
            pltpu.SemaphoreType.DMA,
            pltpu.SemaphoreType.DMA,
        ],
        compiler_params=pltpu.CompilerParams(use_tc_tiling_on_sc=False),
    )(_body)
    out = run(idx, shared_weights)
    return out.reshape(S, T, D)


# speedup vs baseline: 1.0978x; 1.0978x over previous
import functools

import jax
import jax.numpy as jnp
from jax import lax
from jax.experimental import pallas as pl
from jax.experimental.pallas import tpu as pltpu
from jax.experimental.pallas import tpu_sc as plsc

VOCAB = 1000000
D = 32
PACK = 4
SCALE = float(D) ** 0.5

NC = 2
NS = 16
NW = NC * NS

GROUP = 128
NG = 50
G_PER_CHUNK = 2
NCHUNK = NG // G_PER_CHUNK
OROWS = G_PER_CHUNK * GROUP * D // 128


def _body(idx_hbm, table_hbm, out_hbm, idx_v, rows_v, gb0, gb1, ob0, ob1,
          gs0, gs1, ss0, ss1):
    wid = lax.axis_index("s") * NC + lax.axis_index("c")

    pltpu.sync_copy(idx_hbm.at[wid], idx_v)

    def prep(q, _):
        g = q >> 3
        t = (q & 7) * 16
        iv = idx_v[g, pl.ds(t, 16)]
        rows_v[g, pl.ds(t, 16)] = iv >> 2
        return 0

    lax.fori_loop(0, NG * (GROUP // 16), prep, 0)

    gbufs = (gb0, gb1)
    obufs = (ob0, ob1)
    gsems = (gs0, gs1)
    ssems = (ss0, ss1)
    gather_handles = [None, None]
    scatter_handles = [None, None]

    def fire_gather(c):
        b = c % 2
        hs = []
        for j in range(G_PER_CHUNK):
            g = c * G_PER_CHUNK + j
            hs.append(
                pltpu.async_copy(
                    table_hbm.at[rows_v.at[g]], gbufs[b].at[j], gsems[b]
                )
            )
        gather_handles[b] = hs

    def compute_and_scatter(c):
        b = c % 2
        gbuf = gbufs[b]
        obuf = obufs[b]
        for h in gather_handles[b]:
            h.wait()

        def blk(t, _):
            j = t >> 3
            r0 = (t & 7) * 16
            iv = idx_v[c * G_PER_CHUNK + j, pl.ds(r0, 16)]
            offv = (iv & (PACK - 1)) * D
            mv = jnp.where(iv == 0, 0.0, SCALE)
            for r8 in range(16):
                off = offv[r8]
                m = mv[r8]
                src = r0 + r8
                dst = t * 4 + (r8 >> 2)
                lane = (r8 & 3) * D
                v0 = gbuf[j, src, pl.ds(off, 16)] * m
                v1 = gbuf[j, src, pl.ds(off + 16, 16)] * m
                obuf[dst, pl.ds(lane, 16)] = v0
                obuf[dst, pl.ds(lane + 16, 16)] = v1
            return 0

        lax.fori_loop(0, G_PER_CHUNK * (GROUP // 16), blk, 0)
        base = pl.multiple_of(wid * (NG * GROUP * D // 128) + c * OROWS, 8)
        scatter_handles[b] = pltpu.async_copy(
            obuf, out_hbm.at[pl.ds(base, OROWS)], ssems[b]
        )

    fire_gather(0)
    for c in range(NCHUNK):
        if c + 1 < NCHUNK:
            b = (c + 1) % 2
            if scatter_handles[b] is not None:
                scatter_handles[b].wait()
            fire_gather(c + 1)
        compute_and_scatter(c)
    for b in range(2):
        if scatter_handles[b] is not None:
            scatter_handles[b].wait()


@jax.jit
def kernel(inputs, shared_weights):
    S, T = inputs.shape
    n = S * T
    idx = inputs.reshape(NW, NG, GROUP).astype(jnp.int32)
    table = shared_weights.reshape(VOCAB // PACK, PACK * D)

    mesh = plsc.VectorSubcoreMesh(core_axis_name="c", subcore_axis_name="s")
    run = functools.partial(
        pl.kernel,
        mesh=mesh,
        out_type=jax.ShapeDtypeStruct((n * D // 128, 128), jnp.float32),
        scratch_types=[
            pltpu.VMEM((NG, GROUP), jnp.int32),
            pltpu.VMEM((NG, GROUP), jnp.int32),
            pltpu.VMEM((G_PER_CHUNK, GROUP, PACK * D), jnp.float32),
            pltpu.VMEM((G_PER_CHUNK, GROUP, PACK * D), jnp.float32),
            pltpu.VMEM((OROWS, 128), jnp.float32),
            pltpu.VMEM((OROWS, 128), jnp.float32),
            pltpu.SemaphoreType.DMA,
            pltpu.SemaphoreType.DMA,
            pltpu.SemaphoreType.DMA,
            pltpu.SemaphoreType.DMA,
        ],
        compiler_params=pltpu.CompilerParams(use_tc_tiling_on_sc=True),
    )(_body)
    out2 = run(idx, table)
    return out2.reshape(S, T, D)


# speedup vs baseline: 1.2377x; 1.1274x over previous
import functools

import jax
import jax.numpy as jnp
from jax import lax
from jax.experimental import pallas as pl
from jax.experimental.pallas import tpu as pltpu
from jax.experimental.pallas import tpu_sc as plsc

VOCAB = 1000000
D = 32
PACK = 4
SCALE = float(D) ** 0.5

NC = 2
NS = 16
NW = NC * NS

GROUP = 128
NG = 50
G_PER_CHUNK = 2
NCHUNK = NG // G_PER_CHUNK
OROWS = G_PER_CHUNK * GROUP * D // 128


def _body(idx_hbm, table_hbm, out_hbm, idx_v, rows_v, gb0, gb1, ob0, ob1,
          gs0, gs1, ss0, ss1):
    wid = lax.axis_index("s") * NC + lax.axis_index("c")

    pltpu.sync_copy(idx_hbm.at[wid], idx_v)

    def prep(q, _):
        g = q >> 3
        t = (q & 7) * 16
        iv = idx_v[g, pl.ds(t, 16)]
        rows_v[g, pl.ds(t, 16)] = ((iv >> 11) << 9) | (iv & 511)
        return 0

    lax.fori_loop(0, NG * (GROUP // 16), prep, 0)

    gbufs = (gb0, gb1)
    obufs = (ob0, ob1)
    gsems = (gs0, gs1)
    ssems = (ss0, ss1)
    gather_handles = [None, None]
    scatter_handles = [None, None]

    def fire_gather(c):
        b = c % 2
        hs = []
        for j in range(G_PER_CHUNK):
            g = c * G_PER_CHUNK + j
            hs.append(
                pltpu.async_copy(
                    table_hbm.at[rows_v.at[g]], gbufs[b].at[j], gsems[b]
                )
            )
        gather_handles[b] = hs

    def compute_and_scatter(c):
        b = c % 2
        gbuf = gbufs[b]
        obuf = obufs[b]
        for h in gather_handles[b]:
            h.wait()

        def blk(t, _):
            j = t >> 3
            r0 = (t & 7) * 16
            iv = idx_v[c * G_PER_CHUNK + j, pl.ds(r0, 16)]
            offv = ((iv >> 9) & (PACK - 1)) * D
            mv = jnp.where(iv == 0, 0.0, 1.0)
            for r8 in range(16):
                off = offv[r8]
                m = mv[r8]
                src = r0 + r8
                dst = t * 4 + (r8 >> 2)
                lane = (r8 & 3) * D
                v0 = gbuf[j, src, pl.ds(off, 16)] * m
                v1 = gbuf[j, src, pl.ds(off + 16, 16)] * m
                obuf[dst, pl.ds(lane, 16)] = v0
                obuf[dst, pl.ds(lane + 16, 16)] = v1
            return 0

        lax.fori_loop(0, G_PER_CHUNK * (GROUP // 16), blk, 0)
        base = pl.multiple_of(wid * (NG * GROUP * D // 128) + c * OROWS, 8)
        scatter_handles[b] = pltpu.async_copy(
            obuf, out_hbm.at[pl.ds(base, OROWS)], ssems[b]
        )

    fire_gather(0)
    for c in range(NCHUNK):
        if c + 1 < NCHUNK:
            b = (c + 1) % 2
            if scatter_handles[b] is not None:
                scatter_handles[b].wait()
            fire_gather(c + 1)
        compute_and_scatter(c)
    for b in range(2):
        if scatter_handles[b] is not None:
            scatter_handles[b].wait()


TC_COLS = 2048
TC_GRID = -(-VOCAB // TC_COLS)


def _tc_pack_body(w_ref, out_ref):
    x = w_ref[...]
    bands = [x[:, a * 512:(a + 1) * 512].T * SCALE for a in range(PACK)]
    out_ref[...] = jnp.concatenate(bands, axis=1)


def _tc_pack(wT):
    return pl.pallas_call(
        _tc_pack_body,
        grid=(TC_GRID,),
        in_specs=[pl.BlockSpec((D, TC_COLS), lambda i: (0, i))],
        out_specs=pl.BlockSpec((TC_COLS // PACK, PACK * D), lambda i: (i, 0)),
        out_shape=jax.ShapeDtypeStruct(
            (TC_GRID * TC_COLS // PACK, PACK * D), jnp.float32
        ),
    )(wT)


@jax.jit
def kernel(inputs, shared_weights):
    S, T = inputs.shape
    n = S * T
    idx = inputs.reshape(NW, NG, GROUP).astype(jnp.int32)
    table = _tc_pack(shared_weights.T)

    mesh = plsc.VectorSubcoreMesh(core_axis_name="c", subcore_axis_name="s")
    run = functools.partial(
        pl.kernel,
        mesh=mesh,
        out_type=jax.ShapeDtypeStruct((n * D // 128, 128), jnp.float32),
        scratch_types=[
            pltpu.VMEM((NG, GROUP), jnp.int32),
            pltpu.VMEM((NG, GROUP), jnp.int32),
            pltpu.VMEM((G_PER_CHUNK, GROUP, PACK * D), jnp.float32),
            pltpu.VMEM((G_PER_CHUNK, GROUP, PACK * D), jnp.float32),
            pltpu.VMEM((OROWS, 128), jnp.float32),
            pltpu.VMEM((OROWS, 128), jnp.float32),
            pltpu.SemaphoreType.DMA,
            pltpu.SemaphoreType.DMA,
            pltpu.SemaphoreType.DMA,
            pltpu.SemaphoreType.DMA,
        ],
        compiler_params=pltpu.CompilerParams(use_tc_tiling_on_sc=True),
    )(_body)
    out2 = run(idx, table)
    return out2.reshape(S, T, D)


# speedup vs baseline: 1.4541x; 1.1748x over previous
import functools

import jax
import jax.numpy as jnp
from jax import lax
from jax.experimental import pallas as pl
from jax.experimental.pallas import tpu as pltpu
from jax.experimental.pallas import tpu_sc as plsc

VOCAB = 1000000
D = 32
PACK = 4
SCALE = float(D) ** 0.5

NC = 2
NS = 16
NW = NC * NS

GROUP = 128
NG = 50
G_PER_CHUNK = 2
NCHUNK = NG // G_PER_CHUNK
OROWS = G_PER_CHUNK * GROUP * D // 128


def _body(idx_hbm, table_hbm, out_hbm, idx_v, rows_v, gb0, gb1, ob0, ob1,
          gs0, gs1, ss0, ss1):
    wid = lax.axis_index("s") * NC + lax.axis_index("c")

    pltpu.sync_copy(idx_hbm.at[wid], idx_v)

    def prep(q, _):
        g = q >> 3
        t = (q & 7) * 16
        iv = idx_v[g, pl.ds(t, 16)]
        rows_v[g, pl.ds(t, 16)] = ((iv >> 11) << 9) | (iv & 511)
        return 0

    lax.fori_loop(0, NG * (GROUP // 16), prep, 0)

    gbufs = (gb0, gb1)
    obufs = (ob0, ob1)
    gsems = (gs0, gs1)
    ssems = (ss0, ss1)
    gather_handles = [None, None]
    scatter_handles = [None, None]

    def fire_gather(c):
        b = c % 2
        hs = []
        for j in range(G_PER_CHUNK):
            g = c * G_PER_CHUNK + j
            hs.append(
                pltpu.async_copy(
                    table_hbm.at[rows_v.at[g]], gbufs[b].at[j], gsems[b]
                )
            )
        gather_handles[b] = hs

    def compute_and_scatter(c):
        b = c % 2
        gbuf = gbufs[b]
        obuf = obufs[b]
        for h in gather_handles[b]:
            h.wait()

        def blk(t, _):
            j = t >> 3
            r0 = (t & 7) * 16
            iv = idx_v[c * G_PER_CHUNK + j, pl.ds(r0, 16)]
            offv = ((iv >> 9) & (PACK - 1)) * D
            mv = jnp.where(iv == 0, 0.0, 1.0)
            for r8 in range(16):
                off = offv[r8]
                m = mv[r8]
                src = r0 + r8
                dst = t * 4 + (r8 >> 2)
                lane = (r8 & 3) * D
                v0 = gbuf[j, src, pl.ds(off, 16)] * m
                v1 = gbuf[j, src, pl.ds(off + 16, 16)] * m
                obuf[dst, pl.ds(lane, 16)] = v0
                obuf[dst, pl.ds(lane + 16, 16)] = v1
            return 0

        lax.fori_loop(0, G_PER_CHUNK * (GROUP // 16), blk, 0)
        base = pl.multiple_of(wid * (NG * GROUP * D // 128) + c * OROWS, 8)
        scatter_handles[b] = pltpu.async_copy(
            obuf, out_hbm.at[pl.ds(base, OROWS)], ssems[b]
        )

    fire_gather(0)
    for c in range(NCHUNK):
        if c + 1 < NCHUNK:
            b = (c + 1) % 2
            if scatter_handles[b] is not None:
                scatter_handles[b].wait()
            fire_gather(c + 1)
        compute_and_scatter(c)
    for b in range(2):
        if scatter_handles[b] is not None:
            scatter_handles[b].wait()


TC_COLS = 2048
TC_GRID = -(-VOCAB // TC_COLS)


def _tc_pack_body(w_ref, out_ref):
    x = w_ref[...]
    y = jnp.concatenate([x[:, a * 512:(a + 1) * 512] for a in range(PACK)],
                        axis=0)
    out_ref[...] = y.T * SCALE


def _tc_pack(wT):
    return pl.pallas_call(
        _tc_pack_body,
        grid=(TC_GRID,),
        in_specs=[pl.BlockSpec((D, TC_COLS), lambda i: (0, i))],
        out_specs=pl.BlockSpec((TC_COLS // PACK, PACK * D), lambda i: (i, 0)),
        out_shape=jax.ShapeDtypeStruct(
            (TC_GRID * TC_COLS // PACK, PACK * D), jnp.float32
        ),
    )(wT)


@jax.jit
def kernel(inputs, shared_weights):
    S, T = inputs.shape
    n = S * T
    idx = inputs.reshape(NW, NG, GROUP).astype(jnp.int32)
    table = _tc_pack(shared_weights.T)

    mesh = plsc.VectorSubcoreMesh(core_axis_name="c", subcore_axis_name="s")
    run = functools.partial(
        pl.kernel,
        mesh=mesh,
        out_type=jax.ShapeDtypeStruct((n * D // 128, 128), jnp.float32),
        scratch_types=[
            pltpu.VMEM((NG, GROUP), jnp.int32),
            pltpu.VMEM((NG, GROUP), jnp.int32),
            pltpu.VMEM((G_PER_CHUNK, GROUP, PACK * D), jnp.float32),
            pltpu.VMEM((G_PER_CHUNK, GROUP, PACK * D), jnp.float32),
            pltpu.VMEM((OROWS, 128), jnp.float32),
            pltpu.VMEM((OROWS, 128), jnp.float32),
            pltpu.SemaphoreType.DMA,
            pltpu.SemaphoreType.DMA,
            pltpu.SemaphoreType.DMA,
            pltpu.SemaphoreType.DMA,
        ],
        compiler_params=pltpu.CompilerParams(use_tc_tiling_on_sc=True),
    )(_body)
    out2 = run(idx, table)
    return out2.reshape(S, T, D)


# speedup vs baseline: 2.2437x; 1.5430x over previous
import functools

import jax
import jax.numpy as jnp
from jax import lax
from jax.experimental import pallas as pl
from jax.experimental.pallas import tpu as pltpu
from jax.experimental.pallas import tpu_sc as plsc

VOCAB = 1000000
D = 32
PACK = 4
SCALE = float(D) ** 0.5

NC = 2
NS = 16
NW = NC * NS

GROUP = 128
NG = 50
G_PER_CHUNK = 2
NCHUNK = NG // G_PER_CHUNK
OROWS = G_PER_CHUNK * GROUP * D // 128


def _body(idx_hbm, table_hbm, out_hbm, idx_v, rows_v, gb0, gb1, ob0, ob1,
          gs0, gs1, ss0, ss1):
    wid = lax.axis_index("s") * NC + lax.axis_index("c")

    pltpu.sync_copy(idx_hbm.at[wid], idx_v)

    def prep(q, _):
        g = q >> 3
        t = (q & 7) * 16
        iv = idx_v[g, pl.ds(t, 16)]
        rows_v[g, pl.ds(t, 16)] = ((iv >> 13) << 11) | (iv & 2047)
        return 0

    lax.fori_loop(0, NG * (GROUP // 16), prep, 0)

    gbufs = (gb0, gb1)
    obufs = (ob0, ob1)
    gsems = (gs0, gs1)
    ssems = (ss0, ss1)
    gather_handles = [None, None]
    scatter_handles = [None, None]

    def fire_gather(c):
        b = c % 2
        hs = []
        for j in range(G_PER_CHUNK):
            g = c * G_PER_CHUNK + j
            hs.append(
                pltpu.async_copy(
                    table_hbm.at[rows_v.at[g]], gbufs[b].at[j], gsems[b]
                )
            )
        gather_handles[b] = hs

    def compute_and_scatter(c):
        b = c % 2
        gbuf = gbufs[b]
        obuf = obufs[b]
        for h in gather_handles[b]:
            h.wait()

        def blk(t, _):
            j = t >> 3
            r0 = (t & 7) * 16
            iv = idx_v[c * G_PER_CHUNK + j, pl.ds(r0, 16)]
            offv = ((iv >> 11) & (PACK - 1)) * D
            mv = jnp.where(iv == 0, 0.0, 1.0)
            for r8 in range(16):
                off = offv[r8]
                m = mv[r8]
                src = r0 + r8
                dst = t * 4 + (r8 >> 2)
                lane = (r8 & 3) * D
                v0 = gbuf[j, src, pl.ds(off, 16)] * m
                v1 = gbuf[j, src, pl.ds(off + 16, 16)] * m
                obuf[dst, pl.ds(lane, 16)] = v0
                obuf[dst, pl.ds(lane + 16, 16)] = v1
            return 0

        lax.fori_loop(0, G_PER_CHUNK * (GROUP // 16), blk, 0)
        base = pl.multiple_of(wid * (NG * GROUP * D // 128) + c * OROWS, 8)
        scatter_handles[b] = pltpu.async_copy(
            obuf, out_hbm.at[pl.ds(base, OROWS)], ssems[b]
        )

    fire_gather(0)
    for c in range(NCHUNK):
        if c + 1 < NCHUNK:
            b = (c + 1) % 2
            if scatter_handles[b] is not None:
                scatter_handles[b].wait()
            fire_gather(c + 1)
        compute_and_scatter(c)
    for b in range(2):
        if scatter_handles[b] is not None:
            scatter_handles[b].wait()


TC_COLS = 8192
TC_GRID = -(-VOCAB // TC_COLS)


def _tc_pack_body(w_ref, out_ref):
    x = w_ref[...]
    y = jnp.concatenate([x[:, a * (TC_COLS // PACK):(a + 1) * (TC_COLS // PACK)] for a in range(PACK)],
                        axis=0)
    out_ref[...] = y.T * SCALE


def _tc_pack(wT):
    return pl.pallas_call(
        _tc_pack_body,
        grid=(TC_GRID,),
        in_specs=[pl.BlockSpec((D, TC_COLS), lambda i: (0, i))],
        out_specs=pl.BlockSpec((TC_COLS // PACK, PACK * D), lambda i: (i, 0)),
        out_shape=jax.ShapeDtypeStruct(
            (TC_GRID * TC_COLS // PACK, PACK * D), jnp.float32
        ),
    )(wT)


@jax.jit
def kernel(inputs, shared_weights):
    S, T = inputs.shape
    n = S * T
    idx = inputs.reshape(NW, NG, GROUP).astype(jnp.int32)
    table = _tc_pack(shared_weights.T)

    mesh = plsc.VectorSubcoreMesh(core_axis_name="c", subcore_axis_name="s")
    run = functools.partial(
        pl.kernel,
        mesh=mesh,
        out_type=jax.ShapeDtypeStruct((n * D // 128, 128), jnp.float32),
        scratch_types=[
            pltpu.VMEM((NG, GROUP), jnp.int32),
            pltpu.VMEM((NG, GROUP), jnp.int32),
            pltpu.VMEM((G_PER_CHUNK, GROUP, PACK * D), jnp.float32),
            pltpu.VMEM((G_PER_CHUNK, GROUP, PACK * D), jnp.float32),
            pltpu.VMEM((OROWS, 128), jnp.float32),
            pltpu.VMEM((OROWS, 128), jnp.float32),
            pltpu.SemaphoreType.DMA,
            pltpu.SemaphoreType.DMA,
            pltpu.SemaphoreType.DMA,
            pltpu.SemaphoreType.DMA,
        ],
        compiler_params=pltpu.CompilerParams(use_tc_tiling_on_sc=True),
    )(_body)
    out2 = run(idx, table)
    return out2.reshape(S, T, D)


# speedup vs baseline: 2.4333x; 1.0845x over previous
import functools

import jax
import jax.numpy as jnp
from jax import lax
from jax.experimental import pallas as pl
from jax.experimental.pallas import tpu as pltpu
from jax.experimental.pallas import tpu_sc as plsc

VOCAB = 1000000
D = 32
PACK = 4
SCALE = float(D) ** 0.5

NC = 2
NS = 16
NW = NC * NS

GROUP = 128
NG = 50
G_PER_CHUNK = 2
NCHUNK = NG // G_PER_CHUNK
OROWS = G_PER_CHUNK * GROUP * D // 128


def _body(idx_hbm, table_hbm, out_hbm, idx_v, rows_v, gb0, gb1, ob0, ob1,
          gs0, gs1, ss0, ss1):
    wid = lax.axis_index("s") * NC + lax.axis_index("c")

    pltpu.sync_copy(idx_hbm.at[wid], idx_v)

    lane = lax.iota(jnp.int32, 16)

    def prep(q, _):
        g = q >> 3
        t = (q & 7) * 16
        iv = idx_v[g, pl.ds(t, 16)]
        row = ((iv >> 14) << 12) | (iv & (BAND - 1))
        zrow = ZBASE + ((q * 16 + lane) & (BAND - 1))
        rows_v[g, pl.ds(t, 16)] = jnp.where(iv == 0, zrow, row)
        return 0

    lax.fori_loop(0, NG * (GROUP // 16), prep, 0)

    gbufs = (gb0, gb1)
    obufs = (ob0, ob1)
    gsems = (gs0, gs1)
    ssems = (ss0, ss1)
    gather_handles = [None, None]
    scatter_handles = [None, None]

    def fire_gather(c):
        b = c % 2
        hs = []
        for j in range(G_PER_CHUNK):
            g = c * G_PER_CHUNK + j
            hs.append(
                pltpu.async_copy(
                    table_hbm.at[rows_v.at[g]], gbufs[b].at[j], gsems[b]
                )
            )
        gather_handles[b] = hs

    def compute_and_scatter(c):
        b = c % 2
        gbuf = gbufs[b]
        obuf = obufs[b]
        for h in gather_handles[b]:
            h.wait()

        def blk(t, _):
            j = t >> 3
            r0 = (t & 7) * 16
            iv = idx_v[c * G_PER_CHUNK + j, pl.ds(r0, 16)]
            offv = ((iv >> 12) & (PACK - 1)) * D
            for r8 in range(16):
                off = offv[r8]
                src = r0 + r8
                dst = t * 4 + (r8 >> 2)
                lane = (r8 & 3) * D
                obuf[dst, pl.ds(lane, 16)] = gbuf[j, src, pl.ds(off, 16)]
                obuf[dst, pl.ds(lane + 16, 16)] = gbuf[j, src,
                                                       pl.ds(off + 16, 16)]
            return 0

        lax.fori_loop(0, G_PER_CHUNK * (GROUP // 16), blk, 0)
        base = pl.multiple_of(wid * (NG * GROUP * D // 128) + c * OROWS, 8)
        scatter_handles[b] = pltpu.async_copy(
            obuf, out_hbm.at[pl.ds(base, OROWS)], ssems[b]
        )

    fire_gather(0)
    for c in range(NCHUNK):
        if c + 1 < NCHUNK:
            b = (c + 1) % 2
            if scatter_handles[b] is not None:
                scatter_handles[b].wait()
            fire_gather(c + 1)
        compute_and_scatter(c)
    for b in range(2):
        if scatter_handles[b] is not None:
            scatter_handles[b].wait()


TC_COLS = 16384
TC_GRID = -(-VOCAB // TC_COLS)
BAND = TC_COLS // PACK
ZBASE = TC_GRID * BAND


def _tc_pack_body(w_ref, out_ref):
    i = pl.program_id(0)

    @pl.when(i < TC_GRID)
    def _():
        x = w_ref[...]
        y = jnp.concatenate(
            [x[:, a * BAND:(a + 1) * BAND] for a in range(PACK)], axis=0
        )
        out_ref[...] = y.T * SCALE

    @pl.when(i == TC_GRID)
    def _():
        out_ref[...] = jnp.zeros((BAND, PACK * D), jnp.float32)


def _tc_pack(wT):
    return pl.pallas_call(
        _tc_pack_body,
        grid=(TC_GRID + 1,),
        in_specs=[pl.BlockSpec(
            (D, TC_COLS), lambda i: (0, jnp.minimum(i, TC_GRID - 1))
        )],
        out_specs=pl.BlockSpec((BAND, PACK * D), lambda i: (i, 0)),
        out_shape=jax.ShapeDtypeStruct(
            ((TC_GRID + 1) * BAND, PACK * D), jnp.float32
        ),
    )(wT)


@jax.jit
def kernel(inputs, shared_weights):
    S, T = inputs.shape
    n = S * T
    idx = inputs.reshape(NW, NG, GROUP).astype(jnp.int32)
    table = _tc_pack(shared_weights.T)

    mesh = plsc.VectorSubcoreMesh(core_axis_name="c", subcore_axis_name="s")
    run = functools.partial(
        pl.kernel,
        mesh=mesh,
        out_type=jax.ShapeDtypeStruct((n * D // 128, 128), jnp.float32),
        scratch_types=[
            pltpu.VMEM((NG, GROUP), jnp.int32),
            pltpu.VMEM((NG, GROUP), jnp.int32),
            pltpu.VMEM((G_PER_CHUNK, GROUP, PACK * D), jnp.float32),
            pltpu.VMEM((G_PER_CHUNK, GROUP, PACK * D), jnp.float32),
            pltpu.VMEM((OROWS, 128), jnp.float32),
            pltpu.VMEM((OROWS, 128), jnp.float32),
            pltpu.SemaphoreType.DMA,
            pltpu.SemaphoreType.DMA,
            pltpu.SemaphoreType.DMA,
            pltpu.SemaphoreType.DMA,
        ],
        compiler_params=pltpu.CompilerParams(use_tc_tiling_on_sc=True),
    )(_body)
    out2 = run(idx, table)
    return out2.reshape(S, T, D)


# speedup vs baseline: 2.5859x; 1.0627x over previous
import functools

import jax
import jax.numpy as jnp
from jax import lax
from jax.experimental import pallas as pl
from jax.experimental.pallas import tpu as pltpu
from jax.experimental.pallas import tpu_sc as plsc

VOCAB = 1000000
D = 32
PACK = 4
SCALE = float(D) ** 0.5

NC = 2
NS = 16
NW = NC * NS

GROUP = 128
NG = 50
G_PER_CHUNK = 2
NCHUNK = NG // G_PER_CHUNK
OROWS = G_PER_CHUNK * GROUP * D // 128


def _body(idx_hbm, table_hbm, out_hbm, idx_v, rows_v, gb0, gb1, ob0, ob1,
          gs0, gs1, ss0, ss1):
    wid = lax.axis_index("s") * NC + lax.axis_index("c")

    pltpu.sync_copy(idx_hbm.at[:, pl.ds(wid * GROUP, GROUP)], idx_v)

    lane = lax.iota(jnp.int32, 16)

    def prep(q, _):
        g = q >> 3
        t = (q & 7) * 16
        iv = idx_v[g, pl.ds(t, 16)]
        row = ((iv >> 14) << 12) | (iv & (BAND - 1))
        zrow = ZBASE + ((q * 16 + lane) & (BAND - 1))
        rows_v[g, pl.ds(t, 16)] = jnp.where(iv == 0, zrow, row)
        return 0

    lax.fori_loop(0, NG * (GROUP // 16), prep, 0)

    gbufs = (gb0, gb1)
    obufs = (ob0, ob1)
    gsems = (gs0, gs1)
    ssems = (ss0, ss1)
    gather_handles = [None, None]
    scatter_handles = [[None] * G_PER_CHUNK, [None] * G_PER_CHUNK]

    def fire_gather(c):
        b = c % 2
        hs = []
        for j in range(G_PER_CHUNK):
            g = c * G_PER_CHUNK + j
            hs.append(
                pltpu.async_copy(
                    table_hbm.at[rows_v.at[g]],
                    gbufs[b].at[pl.ds(j * GROUP, GROUP)],
                    gsems[b],
                )
            )
        gather_handles[b] = hs

    def compute_and_scatter(c):
        b = c % 2
        gbuf = gbufs[b]
        obuf = obufs[b]
        for h in gather_handles[b]:
            h.wait()

        def sblk(q, _):
            j = q >> 3
            s0 = (q & 7) * 16
            iv = idx_v[c * G_PER_CHUNK + j, pl.ds(s0, 16)]
            offv = ((iv >> 12) & (PACK - 1)) * D
            rows16 = j * GROUP + s0 + lane

            def kblk(k4, _):
                for dk in range(4):
                    k = k4 * 4 + dk
                    vec = plsc.load_gather(gbuf, [rows16, offv + k])
                    obuf[j * D + k, pl.ds(s0, 16)] = vec
                return 0

            lax.fori_loop(0, D // 4, kblk, 0)
            return 0

        lax.fori_loop(0, G_PER_CHUNK * (GROUP // 16), sblk, 0)
        for j in range(G_PER_CHUNK):
            g = c * G_PER_CHUNK + j
            scatter_handles[b][j] = pltpu.async_copy(
                obuf.at[pl.ds(j * D, D)],
                out_hbm.at[g, :, pl.ds(wid * GROUP, GROUP)],
                ssems[b],
            )

    fire_gather(0)
    for c in range(NCHUNK):
        if c + 1 < NCHUNK:
            b = (c + 1) % 2
            for h in scatter_handles[b]:
                if h is not None:
                    h.wait()
            fire_gather(c + 1)
        compute_and_scatter(c)
    for b in range(2):
        for h in scatter_handles[b]:
            if h is not None:
                h.wait()


TC_COLS = 16384
TC_GRID = -(-VOCAB // TC_COLS)
BAND = TC_COLS // PACK
ZBASE = TC_GRID * BAND


def _tc_pack_body(w_ref, out_ref):
    i = pl.program_id(0)

    @pl.when(i < TC_GRID)
    def _():
        x = w_ref[...]
        y = jnp.concatenate(
            [x[:, a * BAND:(a + 1) * BAND] for a in range(PACK)], axis=0
        )
        out_ref[...] = y.T * SCALE

    @pl.when(i == TC_GRID)
    def _():
        out_ref[...] = jnp.zeros((BAND, PACK * D), jnp.float32)


def _tc_pack(wT):
    return pl.pallas_call(
        _tc_pack_body,
        grid=(TC_GRID + 1,),
        in_specs=[pl.BlockSpec(
            (D, TC_COLS), lambda i: (0, jnp.minimum(i, TC_GRID - 1))
        )],
        out_specs=pl.BlockSpec((BAND, PACK * D), lambda i: (i, 0)),
        out_shape=jax.ShapeDtypeStruct(
            ((TC_GRID + 1) * BAND, PACK * D), jnp.float32
        ),
    )(wT)


@jax.jit
def kernel(inputs, shared_weights):
    S, T = inputs.shape
    idx = inputs.T.astype(jnp.int32)
    table = _tc_pack(shared_weights.T)

    mesh = plsc.VectorSubcoreMesh(core_axis_name="c", subcore_axis_name="s")
    run = functools.partial(
        pl.kernel,
        mesh=mesh,
        out_type=jax.ShapeDtypeStruct((T, D, S), jnp.float32),
        scratch_types=[
            pltpu.VMEM((NG, GROUP), jnp.int32),
            pltpu.VMEM((NG, GROUP), jnp.int32),
            pltpu.VMEM((G_PER_CHUNK * GROUP, PACK * D), jnp.float32),
            pltpu.VMEM((G_PER_CHUNK * GROUP, PACK * D), jnp.float32),
            pltpu.VMEM((G_PER_CHUNK * D, GROUP), jnp.float32),
            pltpu.VMEM((G_PER_CHUNK * D, GROUP), jnp.float32),
            pltpu.SemaphoreType.DMA,
            pltpu.SemaphoreType.DMA,
            pltpu.SemaphoreType.DMA,
            pltpu.SemaphoreType.DMA,
        ],
        compiler_params=pltpu.CompilerParams(
            use_tc_tiling_on_sc=True, needs_layout_passes=False
        ),
    )(_body)
    o3 = run(idx, table)
    return jnp.transpose(o3, (2, 0, 1))


# speedup vs baseline: 2.7098x; 1.0479x over previous
import functools

import jax
import jax.numpy as jnp
from jax import lax
from jax.experimental import pallas as pl
from jax.experimental.pallas import tpu as pltpu
from jax.experimental.pallas import tpu_sc as plsc

VOCAB = 1000000
D = 32
PACK = 4
SCALE = float(D) ** 0.5

NC = 2
NS = 16
NW = NC * NS

GROUP = 128
NG = 50
G_PER_CHUNK = 2
NCHUNK = NG // G_PER_CHUNK
OROWS = G_PER_CHUNK * GROUP * D // 128


def _body(idx_hbm, table_hbm, out_hbm, idx_v, rows_v, gb0, gb1, ob0, ob1,
          gs0, gs1, ss0, ss1):
    wid = lax.axis_index("s") * NC + lax.axis_index("c")

    pltpu.sync_copy(idx_hbm.at[:, pl.ds(wid * GROUP, GROUP)], idx_v)

    lane = lax.iota(jnp.int32, 16)

    def prep(q, _):
        g = q >> 3
        t = (q & 7) * 16
        iv = idx_v[g, pl.ds(t, 16)]
        row = ((iv >> 15) << 13) | (iv & (BAND - 1))
        zrow = ZBASE + ((q * 16 + lane) & (BAND - 1))
        rows_v[g, pl.ds(t, 16)] = jnp.where(iv == 0, zrow, row)
        return 0

    lax.fori_loop(0, NG * (GROUP // 16), prep, 0)

    gbufs = (gb0, gb1)
    obufs = (ob0, ob1)
    gsems = (gs0, gs1)
    ssems = (ss0, ss1)
    gather_handles = [None, None]
    scatter_handles = [[None] * G_PER_CHUNK, [None] * G_PER_CHUNK]

    def fire_gather(c):
        b = c % 2
        hs = []
        for j in range(G_PER_CHUNK):
            g = c * G_PER_CHUNK + j
            hs.append(
                pltpu.async_copy(
                    table_hbm.at[rows_v.at[g]],
                    gbufs[b].at[pl.ds(j * GROUP, GROUP)],
                    gsems[b],
                )
            )
        gather_handles[b] = hs

    def compute_and_scatter(c):
        b = c % 2
        gbuf = gbufs[b]
        obuf = obufs[b]
        for h in gather_handles[b]:
            h.wait()

        def sblk(q, _):
            j = q >> 3
            s0 = (q & 7) * 16
            iv = idx_v[c * G_PER_CHUNK + j, pl.ds(s0, 16)]
            offv = ((iv >> 13) & (PACK - 1)) * D
            rows16 = j * GROUP + s0 + lane

            def kblk(k8, _):
                for dk in range(8):
                    k = k8 * 8 + dk
                    vec = plsc.load_gather(gbuf, [rows16, offv + k])
                    obuf[j * D + k, pl.ds(s0, 16)] = vec
                return 0

            lax.fori_loop(0, D // 8, kblk, 0)
            return 0

        lax.fori_loop(0, G_PER_CHUNK * (GROUP // 16), sblk, 0)
        for j in range(G_PER_CHUNK):
            g = c * G_PER_CHUNK + j
            scatter_handles[b][j] = pltpu.async_copy(
                obuf.at[pl.ds(j * D, D)],
                out_hbm.at[g, :, pl.ds(wid * GROUP, GROUP)],
                ssems[b],
            )

    fire_gather(0)
    for c in range(NCHUNK):
        if c + 1 < NCHUNK:
            b = (c + 1) % 2
            for h in scatter_handles[b]:
                if h is not None:
                    h.wait()
            fire_gather(c + 1)
        compute_and_scatter(c)
    for b in range(2):
        for h in scatter_handles[b]:
            if h is not None:
                h.wait()


TC_COLS = 32768
TC_GRID = -(-VOCAB // TC_COLS)
BAND = TC_COLS // PACK
ZBASE = TC_GRID * BAND


def _tc_pack_body(w_ref, out_ref):
    i = pl.program_id(0)

    @pl.when(i < TC_GRID)
    def _():
        x = w_ref[...]
        y = jnp.concatenate(
            [x[:, a * BAND:(a + 1) * BAND] for a in range(PACK)], axis=0
        )
        out_ref[...] = y.T * SCALE

    @pl.when(i == TC_GRID)
    def _():
        out_ref[...] = jnp.zeros((BAND, PACK * D), jnp.float32)


def _tc_pack(wT):
    return pl.pallas_call(
        _tc_pack_body,
        grid=(TC_GRID + 1,),
        in_specs=[pl.BlockSpec(
            (D, TC_COLS), lambda i: (0, jnp.minimum(i, TC_GRID - 1))
        )],
        out_specs=pl.BlockSpec((BAND, PACK * D), lambda i: (i, 0)),
        out_shape=jax.ShapeDtypeStruct(
            ((TC_GRID + 1) * BAND, PACK * D), jnp.float32
        ),
    )(wT)


@jax.jit
def kernel(inputs, shared_weights):
    S, T = inputs.shape
    idx = inputs.T.astype(jnp.int32)
    table = _tc_pack(shared_weights.T)

    mesh = plsc.VectorSubcoreMesh(core_axis_name="c", subcore_axis_name="s")
    run = functools.partial(
        pl.kernel,
        mesh=mesh,
        out_type=jax.ShapeDtypeStruct((T, D, S), jnp.float32),
        scratch_types=[
            pltpu.VMEM((NG, GROUP), jnp.int32),
            pltpu.VMEM((NG, GROUP), jnp.int32),
            pltpu.VMEM((G_PER_CHUNK * GROUP, PACK * D), jnp.float32),
            pltpu.VMEM((G_PER_CHUNK * GROUP, PACK * D), jnp.float32),
            pltpu.VMEM((G_PER_CHUNK * D, GROUP), jnp.float32),
            pltpu.VMEM((G_PER_CHUNK * D, GROUP), jnp.float32),
            pltpu.SemaphoreType.DMA,
            pltpu.SemaphoreType.DMA,
            pltpu.SemaphoreType.DMA,
            pltpu.SemaphoreType.DMA,
        ],
        compiler_params=pltpu.CompilerParams(
            use_tc_tiling_on_sc=True, needs_layout_passes=False
        ),
    )(_body)
    o3 = run(idx, table)
    return jnp.transpose(o3, (2, 0, 1))


# speedup vs baseline: 2.8650x; 1.0573x over previous
import functools

import jax
import jax.numpy as jnp
from jax import lax
from jax.experimental import pallas as pl
from jax.experimental.pallas import tpu as pltpu
from jax.experimental.pallas import tpu_sc as plsc

VOCAB = 1000000
D = 32
PACK = 4
SCALE = float(D) ** 0.5

NC = 2
NS = 16
NW = NC * NS

GROUP = 128
NG = 50
G_PER_CHUNK = 2
NCHUNK = NG // G_PER_CHUNK
OROWS = G_PER_CHUNK * GROUP * D // 128


def _body(idx_hbm, table_hbm, out_hbm, idx_v, rows_v, gb0, gb1, ob0, ob1,
          gs0, gs1, ss0, ss1):
    wid = lax.axis_index("s") * NC + lax.axis_index("c")

    pltpu.sync_copy(idx_hbm.at[:, pl.ds(wid * GROUP, GROUP)], idx_v)

    lane = lax.iota(jnp.int32, 16)

    def prep(q, _):
        g = q >> 3
        t = (q & 7) * 16
        iv = idx_v[g, pl.ds(t, 16)]
        row = ((iv >> 15) << 13) | (iv & (BAND - 1))
        zrow = ZBASE + ((q * 16 + lane) & (BAND - 1))
        rows_v[g, pl.ds(t, 16)] = jnp.where(iv == 0, zrow, row)
        return 0

    lax.fori_loop(0, NG * (GROUP // 16), prep, 0)

    gbufs = (gb0, gb1)
    obufs = (ob0, ob1)
    gsems = (gs0, gs1)
    ssems = (ss0, ss1)
    gather_handles = [None, None]
    scatter_handles = [[None] * G_PER_CHUNK, [None] * G_PER_CHUNK]

    def fire_gather(c):
        b = c % 2
        hs = []
        for j in range(G_PER_CHUNK):
            g = c * G_PER_CHUNK + j
            hs.append(
                pltpu.async_copy(
                    table_hbm.at[rows_v.at[g]],
                    gbufs[b].at[pl.ds(j * GROUP, GROUP)],
                    gsems[b],
                )
            )
        gather_handles[b] = hs

    def compute_and_scatter(c):
        b = c % 2
        gbuf = gbufs[b]
        obuf = obufs[b]
        for h in gather_handles[b]:
            h.wait()

        def sblk(q, _):
            j = q >> 3
            s0 = (q & 7) * 16
            iv = idx_v[c * G_PER_CHUNK + j, pl.ds(s0, 16)]
            offv = ((iv >> 13) & (PACK - 1)) * D
            rows16 = j * GROUP + s0 + lane
            orow16 = j * D + D // 2 + lane

            def kblk(k4, _):
                for dk in range(4):
                    k = k4 * 4 + dk
                    vec = plsc.load_gather(gbuf, [rows16, offv + k])
                    obuf[j * D + k, pl.ds(s0, 16)] = vec
                return 0

            lax.fori_loop(0, D // 8, kblk, 0)
            for r8 in range(16):
                off = offv[r8]
                v = gbuf[j * GROUP + s0 + r8, pl.ds(off + D // 2, 16)]
                col = jnp.full((16,), s0 + r8, jnp.int32)
                plsc.store_scatter(obuf, [orow16, col], v)
            return 0

        lax.fori_loop(0, G_PER_CHUNK * (GROUP // 16), sblk, 0)
        for j in range(G_PER_CHUNK):
            g = c * G_PER_CHUNK + j
            scatter_handles[b][j] = pltpu.async_copy(
                obuf.at[pl.ds(j * D, D)],
                out_hbm.at[g, :, pl.ds(wid * GROUP, GROUP)],
                ssems[b],
            )

    fire_gather(0)
    for c in range(NCHUNK):
        if c + 1 < NCHUNK:
            b = (c + 1) % 2
            for h in scatter_handles[b]:
                if h is not None:
                    h.wait()
            fire_gather(c + 1)
        compute_and_scatter(c)
    for b in range(2):
        for h in scatter_handles[b]:
            if h is not None:
                h.wait()


TC_COLS = 32768
TC_GRID = -(-VOCAB // TC_COLS)
BAND = TC_COLS // PACK
ZBASE = TC_GRID * BAND


def _tc_pack_body(w_ref, out_ref):
    i = pl.program_id(0)

    @pl.when(i < TC_GRID)
    def _():
        x = w_ref[...]
        y = jnp.concatenate(
            [x[:, a * BAND:(a + 1) * BAND] for a in range(PACK)], axis=0
        )
        out_ref[...] = y.T * SCALE

    @pl.when(i == TC_GRID)
    def _():
        out_ref[...] = jnp.zeros((BAND, PACK * D), jnp.float32)


def _tc_pack(wT):
    return pl.pallas_call(
        _tc_pack_body,
        grid=(TC_GRID + 1,),
        in_specs=[pl.BlockSpec(
            (D, TC_COLS), lambda i: (0, jnp.minimum(i, TC_GRID - 1))
        )],
        out_specs=pl.BlockSpec((BAND, PACK * D), lambda i: (i, 0)),
        out_shape=jax.ShapeDtypeStruct(
            ((TC_GRID + 1) * BAND, PACK * D), jnp.float32
        ),
    )(wT)


@jax.jit
def kernel(inputs, shared_weights):
    S, T = inputs.shape
    idx = inputs.T.astype(jnp.int32)
    table = _tc_pack(shared_weights.T)

    mesh = plsc.VectorSubcoreMesh(core_axis_name="c", subcore_axis_name="s")
    run = functools.partial(
        pl.kernel,
        mesh=mesh,
        out_type=jax.ShapeDtypeStruct((T, D, S), jnp.float32),
        scratch_types=[
            pltpu.VMEM((NG, GROUP), jnp.int32),
            pltpu.VMEM((NG, GROUP), jnp.int32),
            pltpu.VMEM((G_PER_CHUNK * GROUP, PACK * D), jnp.float32),
            pltpu.VMEM((G_PER_CHUNK * GROUP, PACK * D), jnp.float32),
            pltpu.VMEM((G_PER_CHUNK * D, GROUP), jnp.float32),
            pltpu.VMEM((G_PER_CHUNK * D, GROUP), jnp.float32),
            pltpu.SemaphoreType.DMA,
            pltpu.SemaphoreType.DMA,
            pltpu.SemaphoreType.DMA,
            pltpu.SemaphoreType.DMA,
        ],
        compiler_params=pltpu.CompilerParams(
            use_tc_tiling_on_sc=True, needs_layout_passes=False
        ),
    )(_body)
    o3 = run(idx, table)
    return jnp.transpose(o3, (2, 0, 1))
